# baseline (speedup 1.0000x reference)
import jax
import jax.numpy as jnp
from jax.experimental import pallas as pl
from jax.experimental.pallas import tpu as pltpu

T = 2048
H = 768
E = 8
TOPK = 2
DFF = 384
NG = 4
TG = 2
RSF = 2.5


def _topk_mask_t(x, k):
    n = x.shape[0]
    rank_rows = []
    for j in range(n):
        xj = x[j:j + 1, :]
        beats = (x > xj).astype(jnp.float32)
        rank_j = jnp.sum(beats, axis=0, keepdims=True)
        if j > 0:
            ties = (x[:j, :] == xj).astype(jnp.float32)
            rank_j = rank_j + jnp.sum(ties, axis=0, keepdims=True)
        rank_rows.append(rank_j)
    rank = jnp.concatenate(rank_rows, axis=0)
    return (rank < k).astype(jnp.float32)


def _routing_weights_t(logits_t, bias_col):
    scores = jax.nn.sigmoid(logits_t)
    s = scores + bias_col
    g = jnp.concatenate(
        [s[2 * j:2 * j + 1, :] + s[2 * j + 1:2 * j + 2, :]
         for j in range(NG)], axis=0)
    sel_g = _topk_mask_t(g, TG)
    mask_e = jnp.concatenate(
        [sel_g[e // (E // NG):e // (E // NG) + 1, :] for e in range(E)],
        axis=0)
    tmp = jnp.where(mask_e > 0.5, s, 0.0)
    sel_e = _topk_mask_t(tmp, TOPK)
    w = scores * sel_e
    return w / jnp.sum(w, axis=0, keepdims=True)


def _swiglu_bf16(gu, scale):
    gate = gu[:, :DFF]
    up = gu[:, DFF:]
    h = gate * jax.nn.sigmoid(gate) * up
    if scale is not None:
        h = h * scale
    return h.astype(jnp.bfloat16)


def _moe_body(x_ref, gw_ref, bias_ref, sguw_ref, wgu_ref, wdt_ref,
              out_ref, w_ref):
    i = pl.program_id(0)

    @pl.when(i == 0)
    def _step0():
        x = x_ref[...]
        logits_t = jax.lax.dot_general(
            gw_ref[...], x, (((1,), (1,)), ((), ())),
            preferred_element_type=jnp.float32)
        sgu = jax.lax.dot_general(
            x, sguw_ref[...], (((1,), (1,)), ((), ())),
            preferred_element_type=jnp.float32)
        w_t = _routing_weights_t(logits_t, bias_ref[...])
        w_ref[...] = w_t.T * RSF
        sh = _swiglu_bf16(sgu, None)
        out_ref[...] = jax.lax.dot_general(
            sh, wdt_ref[...], (((1,), (0,)), ((), ())),
            preferred_element_type=jnp.float32)

    @pl.when(i > 0)
    def _expert():
        x = x_ref[...]
        gu = jax.lax.dot_general(
            x, wgu_ref[...], (((1,), (1,)), ((), ())),
            preferred_element_type=jnp.float32)
        lane = jax.lax.broadcasted_iota(jnp.int32, (1, E), 1)
        onehot = (lane == i - 1).astype(jnp.float32)
        we = jnp.sum(w_ref[...] * onehot, axis=1, keepdims=True)
        h = _swiglu_bf16(gu, we)
        out_ref[...] += jax.lax.dot_general(
            h, wdt_ref[...], (((1,), (0,)), ((), ())),
            preferred_element_type=jnp.float32)


@jax.jit
def kernel(hidden_states, gate_W, e_score_correction_bias, We_gate_up,
           We_down, Ws_gate_up, Ws_down):
    bias_col = e_score_correction_bias.reshape(E, 1)
    wgu2d = We_gate_up.reshape(E * 2 * DFF, H)
    wdt = jnp.concatenate(
        [We_down.swapaxes(1, 2).reshape(E * DFF, H), Ws_down.T],
        axis=0).astype(jnp.bfloat16)

    grid = (E + 1,)
    return pl.pallas_call(
        _moe_body,
        grid=grid,
        in_specs=[
            pl.BlockSpec((T, H), lambda i: (0, 0)),
            pl.BlockSpec((E, H), lambda i: (0, 0)),
            pl.BlockSpec((E, 1), lambda i: (0, 0)),
            pl.BlockSpec((2 * DFF, H), lambda i: (0, 0)),
            pl.BlockSpec((2 * DFF, H),
                         lambda i: (jnp.maximum(i - 1, 0), 0)),
            pl.BlockSpec((DFF, H),
                         lambda i: (jnp.where(i == 0, E, i - 1), 0)),
        ],
        out_specs=pl.BlockSpec((T, H), lambda i: (0, 0)),
        out_shape=jax.ShapeDtypeStruct((T, H), jnp.float32),
        scratch_shapes=[pltpu.VMEM((T, E), jnp.float32)],
        compiler_params=pltpu.CompilerParams(
            dimension_semantics=("arbitrary",),
        ),
    )(hidden_states, gate_W, bias_col, Ws_gate_up, wgu2d, wdt)


# speedup vs baseline: 1.1760x; 1.1760x over previous
import jax
import jax.numpy as jnp
from jax.experimental import pallas as pl
from jax.experimental.pallas import tpu as pltpu

T = 2048
H = 768
E = 8
TOPK = 2
DFF = 384
NG = 4
TG = 2
RSF = 2.5

BT = 256


def _topk_mask_t(x, k):
    n = x.shape[0]
    rank_rows = []
    for j in range(n):
        xj = x[j:j + 1, :]
        beats = (x > xj).astype(jnp.float32)
        rank_j = jnp.sum(beats, axis=0, keepdims=True)
        if j > 0:
            ties = (x[:j, :] == xj).astype(jnp.float32)
            rank_j = rank_j + jnp.sum(ties, axis=0, keepdims=True)
        rank_rows.append(rank_j)
    rank = jnp.concatenate(rank_rows, axis=0)
    return (rank < k).astype(jnp.float32)


def _routing_weights_t(logits_t, bias_col):
    scores = jax.nn.sigmoid(logits_t)
    s = scores + bias_col
    g = jnp.concatenate(
        [s[2 * j:2 * j + 1, :] + s[2 * j + 1:2 * j + 2, :]
         for j in range(NG)], axis=0)
    sel_g = _topk_mask_t(g, TG)
    mask_e = jnp.concatenate(
        [sel_g[e // (E // NG):e // (E // NG) + 1, :] for e in range(E)],
        axis=0)
    tmp = jnp.where(mask_e > 0.5, s, 0.0)
    sel_e = _topk_mask_t(tmp, TOPK)
    w = scores * sel_e
    return w / jnp.sum(w, axis=0, keepdims=True)


def _moe_body(x_ref, gw_ref, bias_ref, wgu_ref, sgu_ref, wd_ref, sd_ref,
              out_ref):
    x = x_ref[...]
    logits_t = jax.lax.dot_general(
        gw_ref[...], x, (((1,), (1,)), ((), ())),
        preferred_element_type=jnp.float32)
    gu_all = jax.lax.dot_general(
        x, wgu_ref[...], (((1,), (1,)), ((), ())),
        preferred_element_type=jnp.float32)
    sgu = jax.lax.dot_general(
        x, sgu_ref[...], (((1,), (1,)), ((), ())),
        preferred_element_type=jnp.float32)

    w_t = _routing_weights_t(logits_t, bias_ref[...])
    w_full = w_t.T * RSF

    sgate = sgu[:, :DFF]
    sup = sgu[:, DFF:]
    sh = (sgate * jax.nn.sigmoid(sgate) * sup).astype(jnp.bfloat16)
    acc = jax.lax.dot_general(
        sh, sd_ref[...], (((1,), (1,)), ((), ())),
        preferred_element_type=jnp.float32)
    for e in range(E):
        gate = gu_all[:, e * 2 * DFF:e * 2 * DFF + DFF]
        up = gu_all[:, e * 2 * DFF + DFF:(e + 1) * 2 * DFF]
        h = (gate * jax.nn.sigmoid(gate) * up
             * w_full[:, e:e + 1]).astype(jnp.bfloat16)
        acc = acc + jax.lax.dot_general(
            h, wd_ref[e], (((1,), (1,)), ((), ())),
            preferred_element_type=jnp.float32)
    out_ref[...] = acc


@jax.jit
def kernel(hidden_states, gate_W, e_score_correction_bias, We_gate_up,
           We_down, Ws_gate_up, Ws_down):
    bias_col = e_score_correction_bias.reshape(E, 1)
    wgu2d = We_gate_up.reshape(E * 2 * DFF, H)

    grid = (T // BT,)
    return pl.pallas_call(
        _moe_body,
        grid=grid,
        in_specs=[
            pl.BlockSpec((BT, H), lambda i: (i, 0)),
            pl.BlockSpec((E, H), lambda i: (0, 0)),
            pl.BlockSpec((E, 1), lambda i: (0, 0)),
            pl.BlockSpec((E * 2 * DFF, H), lambda i: (0, 0)),
            pl.BlockSpec((2 * DFF, H), lambda i: (0, 0)),
            pl.BlockSpec((E, H, DFF), lambda i: (0, 0, 0)),
            pl.BlockSpec((H, DFF), lambda i: (0, 0)),
        ],
        out_specs=pl.BlockSpec((BT, H), lambda i: (i, 0)),
        out_shape=jax.ShapeDtypeStruct((T, H), jnp.float32),
        compiler_params=pltpu.CompilerParams(
            dimension_semantics=("arbitrary",),
        ),
    )(hidden_states, gate_W, bias_col, wgu2d, Ws_gate_up, We_down, Ws_down)
